# baseline (speedup 1.0000x reference)
import jax
import jax.numpy as jnp
from jax import lax
from jax.experimental import pallas as pl
from jax.experimental.pallas import tpu as pltpu
from jax.experimental.pallas import tpu_sc as plsc
import functools

NUM_CLASSES = 81
THRESHOLD = 0.5
NEGPOS_RATIO = 3
VAR0 = 0.1
VAR1 = 0.2

B = 16
P = 16384
G = 64
CHUNK = 2048
NCHUNK = P // CHUNK
RB = 2048
NRB = (B * P) // RB


def _eye4():
    return (lax.broadcasted_iota(jnp.int32, (4, 4), 0) ==
            lax.broadcasted_iota(jnp.int32, (4, 4), 1)).astype(jnp.float32)


def _t4(x):
    return lax.dot_general(_eye4(), x, (((1,), (1,)), ((), ())),
                           preferred_element_type=jnp.float32)


def _match_body(truths_ref, valid_ref, priors_ref, loc_ref,
                pos_ref, lossl_ref, npos_ref, bto_s, bti_s, prs, pfs):
    @pl.when(pl.program_id(0) == 0)
    def _prep():
        def prep_c(c, _):
            pr = _t4(priors_ref[pl.ds(c * CHUNK, CHUNK), :])
            prs[:, pl.ds(c * CHUNK, CHUNK)] = pr
            pcx = pr[0:1]; pcy = pr[1:2]; pw = pr[2:3]; ph = pr[3:4]
            px1 = pcx - pw * 0.5; py1 = pcy - ph * 0.5
            px2 = pcx + pw * 0.5; py2 = pcy + ph * 0.5
            area_p = (px2 - px1) * (py2 - py1)
            pf = jnp.concatenate([px1, py1, px2, py2, area_p], axis=0)
            pfs[:, pl.ds(c * CHUNK, CHUNK)] = pf
            return 0
        lax.fori_loop(0, NCHUNK, prep_c, 0)

    t = truths_ref[0]
    t4 = _t4(t)
    valid = valid_ref[0]
    validc = valid.reshape(G, 1) > 0.5
    tx1 = t[:, 0:1]; ty1 = t[:, 1:2]; tx2 = t[:, 2:3]; ty2 = t[:, 3:4]
    area_t = (tx2 - tx1) * (ty2 - ty1)
    jj = lax.broadcasted_iota(jnp.int32, (G, CHUNK), 0)

    def phase_a(c, carry):
        rm, ri = carry
        pr = _t4(priors_ref[pl.ds(c * CHUNK, CHUNK), :])
        pcx = pr[0:1]; pcy = pr[1:2]; pw = pr[2:3]; ph = pr[3:4]
        px1 = pcx - pw * 0.5; py1 = pcy - ph * 0.5
        px2 = pcx + pw * 0.5; py2 = pcy + ph * 0.5
        area_p = (px2 - px1) * (py2 - py1)
        iw = jnp.maximum(jnp.minimum(tx2, px2) - jnp.maximum(tx1, px1), 0.0)
        ih = jnp.maximum(jnp.minimum(ty2, py2) - jnp.maximum(ty1, py1), 0.0)
        inter = iw * ih
        iou = inter / (area_t + area_p - inter)
        ov = jnp.where(validc, iou, -1.0)
        m = jnp.max(ov, axis=0, keepdims=True)
        bti = jnp.min(jnp.where(ov == m, jj, G), axis=0, keepdims=True)
        bto_s[pl.ds(c, 1), :] = m
        bti_s[pl.ds(c, 1), :] = bti
        pm = jnp.max(ov, axis=1, keepdims=True)
        gidx = lax.broadcasted_iota(jnp.int32, (G, CHUNK), 1) + c * CHUNK
        pidx = jnp.min(jnp.where(ov == pm, gidx, P), axis=1, keepdims=True)
        better = pm > rm
        ri = jnp.where(better, pidx, ri)
        rm = jnp.maximum(rm, pm)
        return rm, ri

    rm0 = jnp.full((G, 1), -jnp.inf, dtype=jnp.float32)
    ri0 = jnp.full((G, 1), P, dtype=jnp.int32)
    _, bpi = lax.fori_loop(0, NCHUNK, phase_a, (rm0, ri0))

    def phase_b(c, carry):
        lacc, nacc = carry
        bto = bto_s[pl.ds(c, 1), :]
        bti = bti_s[pl.ds(c, 1), :]
        gidx = lax.broadcasted_iota(jnp.int32, (G, CHUNK), 1) + c * CHUNK
        matchm = (bpi == gidx) & validc
        fj = jnp.max(jnp.where(matchm, jj, -1), axis=0, keepdims=True)
        forced = fj >= 0
        btif = jnp.where(forced, fj, bti)
        btof = jnp.where(forced, 2.0, bto)
        pos = btof >= THRESHOLD
        posf = pos.astype(jnp.float32)
        oh = (btif == jj).astype(jnp.float32)
        matched = lax.dot_general(t4, oh, (((1,), (0,)), ((), ())),
                                  preferred_element_type=jnp.float32)
        mx1 = matched[0:1]; my1 = matched[1:2]
        mx2 = matched[2:3]; my2 = matched[3:4]
        pr = prs[:, pl.ds(c * CHUNK, CHUNK)]
        pcx = pr[0:1]; pcy = pr[1:2]; pw = pr[2:3]; ph = pr[3:4]
        gcx = ((mx1 + mx2) * 0.5 - pcx) / (VAR0 * pw)
        gcy = ((my1 + my2) * 0.5 - pcy) / (VAR0 * ph)
        gw = jnp.log(jnp.maximum(mx2 - mx1, 1e-30) / pw) / VAR1
        gh = jnp.log(jnp.maximum(my2 - my1, 1e-30) / ph) / VAR1
        ld = _t4(loc_ref[0, pl.ds(c * CHUNK, CHUNK), :])
        dsum = jnp.zeros((1, CHUNK), jnp.float32)
        for gcoord, row in ((gcx, 0), (gcy, 1), (gw, 2), (gh, 3)):
            d = ld[row:row + 1] - gcoord
            ad = jnp.abs(d)
            dsum = dsum + jnp.where(ad < 1.0, 0.5 * d * d, ad - 0.5)
        lacc = lacc + jnp.sum(dsum * posf)
        nacc = nacc + jnp.sum(posf)
        pos_ref[0, pl.ds(c, 1), :] = posf
        return lacc, nacc

    lossl, npos = lax.fori_loop(0, NCHUNK, phase_b, (0.0, 0.0))
    lossl_ref[0, 0, :] = jnp.full((128,), lossl, jnp.float32)
    npos_ref[0, 0, :] = jnp.full((128,), npos, jnp.float32)


def _conf_body(conf_ref, pos_ref, wm_ref, cepos_ref):
    r = pl.program_id(0)
    x = conf_ref[...]
    rowmax = jnp.max(x, axis=1, keepdims=True)
    s = jnp.sum(jnp.exp(x - rowmax), axis=1, keepdims=True)
    lse = jnp.log(s) + rowmax
    cols = lse - x[:, 0:2]
    i2 = (lax.broadcasted_iota(jnp.int32, (2, 2), 0) ==
          lax.broadcasted_iota(jnp.int32, (2, 2), 1)).astype(jnp.float32)
    rows = lax.dot_general(i2, cols, (((1,), (1,)), ((), ())),
                           preferred_element_type=jnp.float32)
    posr = pos_ref[0]
    posb = posr > 0.5
    wm_ref[0] = jnp.where(posb, 0.0, rows[0:1])
    cp = jnp.sum(jnp.where(posb, rows[1:2], 0.0))

    @pl.when(r == 0)
    def _():
        cepos_ref[0, 0, :] = jnp.zeros((128,), jnp.float32)
    cepos_ref[0, 0, :] += jnp.full((128,), cp, jnp.float32)


L = 16


def make_sc_select():
    mesh = plsc.VectorSubcoreMesh(core_axis_name="c", subcore_axis_name="s")

    @functools.partial(
        pl.kernel, mesh=mesh,
        out_type=[
            jax.ShapeDtypeStruct((B, L), jnp.int32),
            jax.ShapeDtypeStruct((B, L), jnp.float32),
            jax.ShapeDtypeStruct((B, L), jnp.float32),
        ],
        scratch_types=[
            pltpu.VMEM((P,), jnp.int32),
            pltpu.VMEM((P,), jnp.float32),
            pltpu.VMEM((L,), jnp.float32),
            pltpu.VMEM((L,), jnp.int32),
            pltpu.VMEM((L,), jnp.float32),
            pltpu.VMEM((L,), jnp.float32),
        ],
    )
    def sc_select(wi_hbm, wf_hbm, k_hbm, v_hbm, cg_hbm, sg_hbm,
                  wiv, wfv, kv, ov, cgv, sgv):
        cid = lax.axis_index("c")
        sid = lax.axis_index("s")
        wid = sid * 2 + cid

        def butterfly_sum(x):
            for sh in (8, 4, 2, 1):
                idx = lax.broadcasted_iota(jnp.int32, (L,), 0) ^ sh
                x = x + jnp.take(x, idx, axis=0)
            return x

        @pl.when(wid < B)
        def _():
            img = wid
            pltpu.sync_copy(wi_hbm.at[img], wiv)
            pltpu.sync_copy(wf_hbm.at[img], wfv)
            pltpu.sync_copy(k_hbm.at[img], kv)
            kvec = kv[...].astype(jnp.int32)

            def search(_, carry):
                lo, hi = carry
                mid = lo + ((hi - lo + 1) >> 1)

                def cnt(i, acc):
                    vb = wiv[pl.ds(i * L, L)]
                    return acc + jnp.where(vb >= mid, 1, 0)

                acc = lax.fori_loop(0, P // L, cnt,
                                    jnp.zeros((L,), jnp.int32), unroll=8)
                pred = butterfly_sum(acc) >= kvec
                return (jnp.where(pred, mid, lo),
                        jnp.where(pred, hi, mid - 1))

            lo, _ = lax.fori_loop(
                0, 31, search,
                (jnp.zeros((L,), jnp.int32),
                 jnp.full((L,), 0x7F800000, jnp.int32)))

            def fin(i, carry):
                cg, sg = carry
                vb = wiv[pl.ds(i * L, L)]
                vf = wfv[pl.ds(i * L, L)]
                gt = vb > lo
                return (cg + jnp.where(gt, 1.0, 0.0),
                        sg + jnp.where(gt, vf, 0.0))

            cg, sg = lax.fori_loop(
                0, P // L, fin,
                (jnp.zeros((L,), jnp.float32), jnp.zeros((L,), jnp.float32)),
                unroll=8)
            ov[...] = lo
            cgv[...] = cg
            sgv[...] = sg
            pltpu.sync_copy(ov, v_hbm.at[img])
            pltpu.sync_copy(cgv, cg_hbm.at[img])
            pltpu.sync_copy(sgv, sg_hbm.at[img])

    return sc_select


_SC_SELECT = make_sc_select()


@jax.jit
def kernel(loc_data, conf_data, priors, gt_bboxes, gt_labels, gt_num,
           img_shape):
    del gt_labels
    hw = img_shape.astype(jnp.float32)
    scale = jnp.stack([hw[:, 1], hw[:, 0], hw[:, 1], hw[:, 0]], axis=1)
    truths = gt_bboxes / scale[:, None, :]
    valid = (jnp.arange(G)[None, :] < gt_num[:, None]).astype(jnp.float32)
    valid = valid.reshape(B, 1, G)

    pos, lossl, npos = pl.pallas_call(
        _match_body,
        grid=(B,),
        in_specs=[
            pl.BlockSpec((1, G, 4), lambda b: (b, 0, 0)),
            pl.BlockSpec((1, 1, G), lambda b: (b, 0, 0)),
            pl.BlockSpec((P, 4), lambda b: (0, 0)),
            pl.BlockSpec((1, P, 4), lambda b: (b, 0, 0)),
        ],
        out_specs=[
            pl.BlockSpec((1, NCHUNK, CHUNK), lambda b: (b, 0, 0)),
            pl.BlockSpec((1, 1, 128), lambda b: (b, 0, 0)),
            pl.BlockSpec((1, 1, 128), lambda b: (b, 0, 0)),
        ],
        out_shape=[
            jax.ShapeDtypeStruct((B, NCHUNK, CHUNK), jnp.float32),
            jax.ShapeDtypeStruct((B, 1, 128), jnp.float32),
            jax.ShapeDtypeStruct((B, 1, 128), jnp.float32),
        ],
        scratch_shapes=[
            pltpu.VMEM((NCHUNK, CHUNK), jnp.float32),
            pltpu.VMEM((NCHUNK, CHUNK), jnp.int32),
            pltpu.VMEM((4, P), jnp.float32),
            pltpu.VMEM((5, P), jnp.float32),
        ],
    )(truths, valid, priors, loc_data)

    conf_rows = conf_data.reshape(B * P, NUM_CLASSES)
    wm, cepos = pl.pallas_call(
        _conf_body,
        grid=(NRB,),
        in_specs=[
            pl.BlockSpec((RB, NUM_CLASSES), lambda r: (r, 0)),
            pl.BlockSpec((1, 1, RB), lambda r: (r, 0, 0)),
        ],
        out_specs=[
            pl.BlockSpec((1, 1, RB), lambda r: (r, 0, 0)),
            pl.BlockSpec((1, 1, 128), lambda r: (0, 0, 0)),
        ],
        out_shape=[
            jax.ShapeDtypeStruct((NRB, 1, RB), jnp.float32),
            jax.ShapeDtypeStruct((1, 1, 128), jnp.float32),
        ],
    )(conf_rows, pos.reshape(NRB, 1, RB))

    npos_i = npos[:, 0, 0]
    kf = jnp.minimum(NEGPOS_RATIO * npos_i, float(P - 1))
    krep = jnp.broadcast_to(kf[:, None], (B, 16))
    wflat = wm.reshape(B, P)
    vb, cg, sg = _SC_SELECT(
        lax.bitcast_convert_type(wflat, jnp.int32), wflat, krep)
    v0 = lax.bitcast_convert_type(vb[:, 0], jnp.float32)
    cnt_gt = jnp.sum(cg, axis=1)
    sum_gt = jnp.sum(sg, axis=1)
    neg = sum_gt + jnp.where(v0 > 0.0, (kf - cnt_gt) * v0, 0.0)
    n_total = jnp.maximum(jnp.sum(npos_i), 1.0)
    loss_l = jnp.sum(lossl[:, 0, 0]) / n_total
    loss_c = (cepos[0, 0, 0] + jnp.sum(neg)) / n_total
    return loss_l, loss_c


# speedup vs baseline: 1.0849x; 1.0849x over previous
import jax
import jax.numpy as jnp
from jax import lax
from jax.experimental import pallas as pl
from jax.experimental.pallas import tpu as pltpu

NUM_CLASSES = 81
THRESHOLD = 0.5
NEGPOS_RATIO = 3
VAR0 = 0.1
VAR1 = 0.2

B = 16
P = 16384
G = 64
CHUNK = 2048
NCHUNK = P // CHUNK
RB = 2048
NRB = (B * P) // RB


def _eye4():
    return (lax.broadcasted_iota(jnp.int32, (4, 4), 0) ==
            lax.broadcasted_iota(jnp.int32, (4, 4), 1)).astype(jnp.float32)


def _t4(x):
    return lax.dot_general(_eye4(), x, (((1,), (1,)), ((), ())),
                           preferred_element_type=jnp.float32)


def _match_body(truths_ref, valid_ref, priors_ref, loc_ref,
                pos_ref, lossl_ref, bto_s, bti_s, prs, pfs):
    @pl.when(pl.program_id(0) == 0)
    def _prep():
        def prep_c(c, _):
            pr = _t4(priors_ref[pl.ds(c * CHUNK, CHUNK), :])
            prs[:, pl.ds(c * CHUNK, CHUNK)] = pr
            pcx = pr[0:1]; pcy = pr[1:2]; pw = pr[2:3]; ph = pr[3:4]
            px1 = pcx - pw * 0.5; py1 = pcy - ph * 0.5
            px2 = pcx + pw * 0.5; py2 = pcy + ph * 0.5
            area_p = (px2 - px1) * (py2 - py1)
            pf = jnp.concatenate([px1, py1, px2, py2, area_p], axis=0)
            pfs[:, pl.ds(c * CHUNK, CHUNK)] = pf
            return 0
        lax.fori_loop(0, NCHUNK, prep_c, 0)

    t = truths_ref[0]
    t4 = _t4(t)
    valid = valid_ref[0]
    validc = valid.reshape(G, 1) > 0.5
    tx1 = t[:, 0:1]; ty1 = t[:, 1:2]; tx2 = t[:, 2:3]; ty2 = t[:, 3:4]
    area_t = (tx2 - tx1) * (ty2 - ty1)
    jj = lax.broadcasted_iota(jnp.int32, (G, CHUNK), 0)

    def phase_a(c, carry):
        rm, ri = carry
        pr = _t4(priors_ref[pl.ds(c * CHUNK, CHUNK), :])
        pcx = pr[0:1]; pcy = pr[1:2]; pw = pr[2:3]; ph = pr[3:4]
        px1 = pcx - pw * 0.5; py1 = pcy - ph * 0.5
        px2 = pcx + pw * 0.5; py2 = pcy + ph * 0.5
        area_p = (px2 - px1) * (py2 - py1)
        iw = jnp.maximum(jnp.minimum(tx2, px2) - jnp.maximum(tx1, px1), 0.0)
        ih = jnp.maximum(jnp.minimum(ty2, py2) - jnp.maximum(ty1, py1), 0.0)
        inter = iw * ih
        iou = inter / (area_t + area_p - inter)
        ov = jnp.where(validc, iou, -1.0)
        m = jnp.max(ov, axis=0, keepdims=True)
        bti = jnp.min(jnp.where(ov == m, jj, G), axis=0, keepdims=True)
        bto_s[pl.ds(c, 1), :] = m
        bti_s[pl.ds(c, 1), :] = bti
        pm = jnp.max(ov, axis=1, keepdims=True)
        gidx = lax.broadcasted_iota(jnp.int32, (G, CHUNK), 1) + c * CHUNK
        pidx = jnp.min(jnp.where(ov == pm, gidx, P), axis=1, keepdims=True)
        better = pm > rm
        ri = jnp.where(better, pidx, ri)
        rm = jnp.maximum(rm, pm)
        return rm, ri

    rm0 = jnp.full((G, 1), -jnp.inf, dtype=jnp.float32)
    ri0 = jnp.full((G, 1), P, dtype=jnp.int32)
    _, bpi = lax.fori_loop(0, NCHUNK, phase_a, (rm0, ri0))

    def phase_b(c, lacc):
        bto = bto_s[pl.ds(c, 1), :]
        bti = bti_s[pl.ds(c, 1), :]
        gidx = lax.broadcasted_iota(jnp.int32, (G, CHUNK), 1) + c * CHUNK
        matchm = (bpi == gidx) & validc
        fj = jnp.max(jnp.where(matchm, jj, -1), axis=0, keepdims=True)
        forced = fj >= 0
        btif = jnp.where(forced, fj, bti)
        btof = jnp.where(forced, 2.0, bto)
        pos = btof >= THRESHOLD
        posf = pos.astype(jnp.float32)
        oh = (btif == jj).astype(jnp.float32)
        matched = lax.dot_general(t4, oh, (((1,), (0,)), ((), ())),
                                  preferred_element_type=jnp.float32)
        mx1 = matched[0:1]; my1 = matched[1:2]
        mx2 = matched[2:3]; my2 = matched[3:4]
        pr = prs[:, pl.ds(c * CHUNK, CHUNK)]
        pcx = pr[0:1]; pcy = pr[1:2]; pw = pr[2:3]; ph = pr[3:4]
        gcx = ((mx1 + mx2) * 0.5 - pcx) / (VAR0 * pw)
        gcy = ((my1 + my2) * 0.5 - pcy) / (VAR0 * ph)
        gw = jnp.log(jnp.maximum(mx2 - mx1, 1e-30) / pw) / VAR1
        gh = jnp.log(jnp.maximum(my2 - my1, 1e-30) / ph) / VAR1
        ld = _t4(loc_ref[0, pl.ds(c * CHUNK, CHUNK), :])
        dsum = jnp.zeros((1, CHUNK), jnp.float32)
        for gcoord, row in ((gcx, 0), (gcy, 1), (gw, 2), (gh, 3)):
            d = ld[row:row + 1] - gcoord
            ad = jnp.abs(d)
            dsum = dsum + jnp.where(ad < 1.0, 0.5 * d * d, ad - 0.5)
        lacc = lacc + jnp.sum(dsum * posf)
        pos_ref[0, pl.ds(c, 1), :] = posf
        return lacc

    lossl = lax.fori_loop(0, NCHUNK, phase_b, 0.0)
    lossl_ref[0, 0, :] = jnp.full((128,), lossl, jnp.float32)


def _conf_body(conf_ref, w0_ref, ce1_ref):
    x = conf_ref[...]
    rowmax = jnp.max(x, axis=1, keepdims=True)
    s = jnp.sum(jnp.exp(x - rowmax), axis=1, keepdims=True)
    lse = jnp.log(s) + rowmax
    cols = lse - x[:, 0:2]
    i2 = (lax.broadcasted_iota(jnp.int32, (2, 2), 0) ==
          lax.broadcasted_iota(jnp.int32, (2, 2), 1)).astype(jnp.float32)
    rows = lax.dot_general(i2, cols, (((1,), (1,)), ((), ())),
                           preferred_element_type=jnp.float32)
    c = pl.program_id(0) % NCHUNK
    w0_ref[0, pl.ds(c, 1), :] = rows[0:1]
    ce1_ref[0, pl.ds(c, 1), :] = rows[1:2]


def _select_body(w0_ref, ce1_ref, pos_ref, lossl_ref, out_ref):
    posf = pos_ref[...]
    posb = posf > 0.5
    w = jnp.where(posb, 0.0, w0_ref[...])
    cepos = jnp.sum(jnp.where(posb, ce1_ref[...], 0.0))
    npos = jnp.sum(posf, axis=(1, 2), keepdims=True)
    k = jnp.minimum(NEGPOS_RATIO * npos, float(P - 1))
    ki = k.astype(jnp.int32)
    wb = lax.bitcast_convert_type(w, jnp.int32)

    def step(_, carry):
        lo, hi = carry
        mid = lo + ((hi - lo + 1) >> 1)
        cnt = jnp.sum((wb >= mid).astype(jnp.int32), axis=(1, 2),
                      keepdims=True)
        pred = cnt >= ki
        lo = jnp.where(pred, mid, lo)
        hi = jnp.where(pred, hi, mid - 1)
        return lo, hi

    lo0 = jnp.zeros((B, 1, 1), jnp.int32)
    hi0 = jnp.full((B, 1, 1), 0x7F800000, jnp.int32)
    vb, _ = lax.fori_loop(0, 31, step, (lo0, hi0))
    v = lax.bitcast_convert_type(vb, jnp.float32)
    gt = w > v
    cnt_gt = jnp.sum(gt.astype(jnp.float32), axis=(1, 2), keepdims=True)
    sum_gt = jnp.sum(jnp.where(gt, w, 0.0), axis=(1, 2), keepdims=True)
    neg = sum_gt + jnp.where(v > 0.0, (k - cnt_gt) * v, 0.0)
    n_total = jnp.maximum(jnp.sum(npos), 1.0)
    loss_l = jnp.sum(lossl_ref[:, 0, 0:1]) / n_total
    loss_c = (cepos + jnp.sum(neg)) / n_total
    out_ref[0, 0, :] = jnp.full((128,), loss_l, jnp.float32)
    out_ref[1, 0, :] = jnp.full((128,), loss_c, jnp.float32)


@jax.jit
def kernel(loc_data, conf_data, priors, gt_bboxes, gt_labels, gt_num,
           img_shape):
    del gt_labels
    hw = img_shape.astype(jnp.float32)
    scale = jnp.stack([hw[:, 1], hw[:, 0], hw[:, 1], hw[:, 0]], axis=1)
    truths = gt_bboxes / scale[:, None, :]
    valid = (jnp.arange(G)[None, :] < gt_num[:, None]).astype(jnp.float32)
    valid = valid.reshape(B, 1, G)

    pos, lossl = pl.pallas_call(
        _match_body,
        grid=(B,),
        in_specs=[
            pl.BlockSpec((1, G, 4), lambda b: (b, 0, 0)),
            pl.BlockSpec((1, 1, G), lambda b: (b, 0, 0)),
            pl.BlockSpec((P, 4), lambda b: (0, 0)),
            pl.BlockSpec((1, P, 4), lambda b: (b, 0, 0)),
        ],
        out_specs=[
            pl.BlockSpec((1, NCHUNK, CHUNK), lambda b: (b, 0, 0)),
            pl.BlockSpec((1, 1, 128), lambda b: (b, 0, 0)),
        ],
        out_shape=[
            jax.ShapeDtypeStruct((B, NCHUNK, CHUNK), jnp.float32),
            jax.ShapeDtypeStruct((B, 1, 128), jnp.float32),
        ],
        scratch_shapes=[
            pltpu.VMEM((NCHUNK, CHUNK), jnp.float32),
            pltpu.VMEM((NCHUNK, CHUNK), jnp.int32),
            pltpu.VMEM((4, P), jnp.float32),
            pltpu.VMEM((5, P), jnp.float32),
        ],
    )(truths, valid, priors, loc_data)

    conf_rows = conf_data.reshape(B * P, NUM_CLASSES)
    w0, ce1 = pl.pallas_call(
        _conf_body,
        grid=(NRB,),
        in_specs=[pl.BlockSpec((RB, NUM_CLASSES), lambda r: (r, 0))],
        out_specs=[
            pl.BlockSpec((1, NCHUNK, CHUNK), lambda r: (r // NCHUNK, 0, 0)),
            pl.BlockSpec((1, NCHUNK, CHUNK), lambda r: (r // NCHUNK, 0, 0)),
        ],
        out_shape=[
            jax.ShapeDtypeStruct((B, NCHUNK, CHUNK), jnp.float32),
            jax.ShapeDtypeStruct((B, NCHUNK, CHUNK), jnp.float32),
        ],
    )(conf_rows)

    shp = (B, NCHUNK, CHUNK)
    out = pl.pallas_call(
        _select_body,
        in_specs=[
            pl.BlockSpec(shp, lambda: (0, 0, 0)),
            pl.BlockSpec(shp, lambda: (0, 0, 0)),
            pl.BlockSpec(shp, lambda: (0, 0, 0)),
            pl.BlockSpec((B, 1, 128), lambda: (0, 0, 0)),
        ],
        out_specs=pl.BlockSpec((2, 1, 128), lambda: (0, 0, 0)),
        out_shape=jax.ShapeDtypeStruct((2, 1, 128), jnp.float32),
    )(w0, ce1, pos, lossl)

    return out[0, 0, 0], out[1, 0, 0]
